# Initial kernel scaffold; baseline (speedup 1.0000x reference)
import jax
import jax.numpy as jnp
from jax.experimental import pallas as pl


def kernel(x_cat, tables):
    raise NotImplementedError("write your pallas kernel here")


# speedup vs baseline: 1.2060x; 1.2060x over previous
import functools

import jax
import jax.numpy as jnp
import numpy as np
from jax import lax
from jax.experimental import pallas as pl
from jax.experimental.pallas import tpu as pltpu
from jax.experimental.pallas import tpu_sc as plsc

NC = 2
NS = 16
L = 16
NW = NC * NS

IDX_W = 128
SUB = 13
CHUNK = SUB * IDX_W


@functools.lru_cache(maxsize=None)
def _build(B, F, V, D):
    TOT = B * F
    assert TOT % (NW * CHUNK) == 0
    per_w = TOT // NW
    n_chunks = per_w // CHUNK

    mesh = plsc.VectorSubcoreMesh(core_axis_name="c", subcore_axis_name="s")

    @functools.partial(
        pl.kernel,
        mesh=mesh,
        out_type=jax.ShapeDtypeStruct((TOT, D), jnp.float32),
        scratch_types=[
            pltpu.VMEM((CHUNK,), jnp.int32),
            pltpu.VMEM((CHUNK,), jnp.int32),
            pltpu.VMEM((CHUNK, D), jnp.float32),
            pltpu.SemaphoreType.DMA,
        ],
        compiler_params=pltpu.CompilerParams(use_tc_tiling_on_sc=False),
    )
    def gather_kernel(x_hbm, tab_hbm, offs_hbm, out_hbm, idx_v, offs_v,
                      rows_v, sem):
        wid = lax.axis_index("s") * NC + lax.axis_index("c")
        pltpu.sync_copy(offs_hbm, offs_v)

        def chunk_body(c, carry):
            base = pl.multiple_of((wid * n_chunks + c) * CHUNK, 8)
            pltpu.sync_copy(x_hbm.at[pl.ds(base, CHUNK)], idx_v)
            for t in range(CHUNK // L):
                sl = pl.ds(t * L, L)
                idx_v[sl] = idx_v[sl] + offs_v[sl]
            copies = [
                pltpu.async_copy(
                    tab_hbm.at[idx_v.at[pl.ds(k * IDX_W, IDX_W)]],
                    rows_v.at[pl.ds(k * IDX_W, IDX_W)],
                    sem,
                )
                for k in range(SUB)
            ]
            for cp in copies:
                cp.wait()
            pltpu.sync_copy(rows_v, out_hbm.at[pl.ds(base, CHUNK)])
            return carry

        lax.fori_loop(0, n_chunks, chunk_body, None)

    return gather_kernel


@functools.lru_cache(maxsize=None)
def _field_offsets(F, V):
    offs = (np.arange(CHUNK, dtype=np.int64) % F) * V
    return jnp.asarray(offs.astype(np.int32))


def kernel(x_cat, tables):
    B, F = x_cat.shape
    _, V, D = tables.shape
    x_flat = x_cat.reshape(-1)
    tab = tables.reshape(F * V, D)
    out = _build(B, F, V, D)(x_flat, tab, _field_offsets(F, V))
    return out.reshape(B, F * D)
